# Initial kernel scaffold; baseline (speedup 1.0000x reference)
import jax
import jax.numpy as jnp
from jax.experimental import pallas as pl


def kernel(x, table_01, table_12):
    raise NotImplementedError("write your pallas kernel here")


# speedup vs baseline: 253.9961x; 253.9961x over previous
import functools

import jax
import jax.numpy as jnp
from jax import lax
from jax.experimental import pallas as pl
from jax.experimental.pallas import tpu as pltpu
from jax.experimental.pallas import tpu_sc as plsc

L = 16
NC = 2
NS = 16
NW = NC * NS

N_ROWS, N_COLS = 16384, 200
N_TOTAL = N_ROWS * N_COLS
PER_W = N_TOTAL // NW
CHUNK = 4096
N_CHUNKS = PER_W // CHUNK
T01_SIZE = 100000
T12_SIZE = 1000


def _remap_body(x_hbm, t01_hbm, t12_hbm, out_hbm, t01_v, t12_v, xb, ob):
    wid = lax.axis_index("s") * NC + lax.axis_index("c")
    base = wid * PER_W
    pltpu.sync_copy(t01_hbm, t01_v)
    pltpu.sync_copy(t12_hbm, t12_v)

    def chunk_body(c, carry):
        off = base + c * CHUNK
        pltpu.sync_copy(x_hbm.at[pl.ds(off, CHUNK)], xb)

        def vec_body(i, carry2):
            v = xb[pl.ds(i * L, L)]
            mid = plsc.load_gather(t01_v, [v])
            fin = plsc.load_gather(t12_v, [mid])
            ob[pl.ds(i * L, L)] = fin
            return carry2

        lax.fori_loop(0, CHUNK // L, vec_body, 0, unroll=4)
        pltpu.sync_copy(ob, out_hbm.at[pl.ds(off, CHUNK)])
        return carry

    lax.fori_loop(0, N_CHUNKS, chunk_body, 0)


@jax.jit
def _remap(xf, table_01, table_12):
    mesh = plsc.VectorSubcoreMesh(core_axis_name="c", subcore_axis_name="s")
    return pl.kernel(
        _remap_body,
        mesh=mesh,
        out_type=jax.ShapeDtypeStruct((N_TOTAL,), jnp.int32),
        scratch_types=[
            pltpu.VMEM((T01_SIZE,), jnp.int32),
            pltpu.VMEM((T12_SIZE,), jnp.int32),
            pltpu.VMEM((CHUNK,), jnp.int32),
            pltpu.VMEM((CHUNK,), jnp.int32),
        ],
        compiler_params=pltpu.CompilerParams(needs_layout_passes=False),
    )(xf, table_01, table_12)


def kernel(x, table_01, table_12):
    out = _remap(x.reshape(-1), table_01, table_12)
    return out.reshape(N_ROWS, N_COLS)


# speedup vs baseline: 444.2098x; 1.7489x over previous
import jax
import jax.numpy as jnp
from jax import lax
from jax.experimental import pallas as pl
from jax.experimental.pallas import tpu as pltpu
from jax.experimental.pallas import tpu_sc as plsc

L = 16
NC = 2
NS = 16
NW = NC * NS

N_ROWS, N_COLS = 16384, 200
N_TOTAL = N_ROWS * N_COLS
PER_W = N_TOTAL // NW
CHUNK = 5120
N_CHUNKS = PER_W // CHUNK
T01_SIZE = 100000
T12_SIZE = 1000


def _remap_body(x_hbm, t01_hbm, t12_hbm, out_hbm, t01_v, t12_v,
                xb0, xb1, ob0, ob1, tsem, is0, is1, os0, os1):
    wid = lax.axis_index("s") * NC + lax.axis_index("c")
    base = wid * PER_W
    t01_cp = pltpu.async_copy(t01_hbm, t01_v, tsem)
    t12_cp = pltpu.async_copy(t12_hbm, t12_v, tsem)

    xb = [xb0, xb1]
    ob = [ob0, ob1]
    isem = [is0, is1]
    osem = [os0, os1]
    in_h = [None] * N_CHUNKS
    out_h = [None] * N_CHUNKS

    def start_in(c):
        in_h[c] = pltpu.async_copy(
            x_hbm.at[pl.ds(base + c * CHUNK, CHUNK)], xb[c % 2], isem[c % 2])

    def start_out(c):
        out_h[c] = pltpu.async_copy(
            ob[c % 2], out_hbm.at[pl.ds(base + c * CHUNK, CHUNK)], osem[c % 2])

    start_in(0)
    start_in(1)
    t01_cp.wait()
    t12_cp.wait()

    for c in range(N_CHUNKS):
        in_h[c].wait()
        if c >= 2:
            out_h[c - 2].wait()
        xbc = xb[c % 2]
        obc = ob[c % 2]

        @plsc.parallel_loop(0, CHUNK, L, unroll=8)
        def _gather(i):
            v = xbc[pl.ds(i, L)]
            mid = plsc.load_gather(t01_v, [v])
            obc[pl.ds(i, L)] = plsc.load_gather(t12_v, [mid])

        start_out(c)
        if c + 2 < N_CHUNKS:
            start_in(c + 2)

    out_h[N_CHUNKS - 2].wait()
    out_h[N_CHUNKS - 1].wait()


@jax.jit
def _remap(xf, table_01, table_12):
    mesh = plsc.VectorSubcoreMesh(core_axis_name="c", subcore_axis_name="s")
    return pl.kernel(
        _remap_body,
        mesh=mesh,
        out_type=jax.ShapeDtypeStruct((N_TOTAL,), jnp.int32),
        scratch_types=[
            pltpu.VMEM((T01_SIZE,), jnp.int32),
            pltpu.VMEM((T12_SIZE,), jnp.int32),
            pltpu.VMEM((CHUNK,), jnp.int32),
            pltpu.VMEM((CHUNK,), jnp.int32),
            pltpu.VMEM((CHUNK,), jnp.int32),
            pltpu.VMEM((CHUNK,), jnp.int32),
            pltpu.SemaphoreType.DMA,
            pltpu.SemaphoreType.DMA,
            pltpu.SemaphoreType.DMA,
            pltpu.SemaphoreType.DMA,
            pltpu.SemaphoreType.DMA,
        ],
        compiler_params=pltpu.CompilerParams(needs_layout_passes=False),
    )(xf, table_01, table_12)


def kernel(x, table_01, table_12):
    out = _remap(x.reshape(-1), table_01, table_12)
    return out.reshape(N_ROWS, N_COLS)


# speedup vs baseline: 669.8299x; 1.5079x over previous
import jax
import jax.numpy as jnp
from jax import lax
from jax.experimental import pallas as pl
from jax.experimental.pallas import tpu as pltpu
from jax.experimental.pallas import tpu_sc as plsc

L = 16
NC = 2
NS = 16
NW = NC * NS

N_ROWS, N_COLS = 16384, 200
ROWS_PER_W = N_ROWS // NW
RCHUNK = 16
N_CHUNKS = ROWS_PER_W // RCHUNK
N_PAIRS = N_CHUNKS // 2
T01_SIZE = 100000
T12_SIZE = 1000
COL_OFFS = tuple(range(0, 192, 16)) + (N_COLS - L,)


def _remap_body(x_hbm, t01_hbm, t12_hbm, out_hbm, t01_v, t12_v,
                xb0, xb1, ob0, ob1, tsem, is0, is1, os0, os1):
    wid = lax.axis_index("s") * NC + lax.axis_index("c")
    base = wid * ROWS_PER_W
    t01_cp = pltpu.async_copy(t01_hbm, t01_v, tsem)
    t12_cp = pltpu.async_copy(t12_hbm, t12_v, tsem)

    xb = [xb0, xb1]
    ob = [ob0, ob1]
    isem = [is0, is1]
    osem = [os0, os1]

    def in_copy(c, b):
        return pltpu.make_async_copy(
            x_hbm.at[pl.ds(base + c * RCHUNK, RCHUNK)], xb[b], isem[b])

    def out_copy(c, b):
        return pltpu.make_async_copy(
            ob[b], out_hbm.at[pl.ds(base + c * RCHUNK, RCHUNK)], osem[b])

    def compute(b):
        xbc = xb[b]
        obc = ob[b]

        @plsc.parallel_loop(0, RCHUNK, 1, unroll=2)
        def _row(r):
            for c0 in COL_OFFS:
                v = xbc[r, pl.ds(c0, L)]
                mid = plsc.load_gather(t01_v, [v])
                obc[r, pl.ds(c0, L)] = plsc.load_gather(t12_v, [mid])

    def do_chunk(c, b, wait_out, start_next):
        in_copy(c, b).wait()
        if wait_out:
            out_copy(c - 2, b).wait()
        compute(b)
        out_copy(c, b).start()
        if start_next:
            in_copy(c + 2, b).start()

    in_copy(0, 0).start()
    in_copy(1, 1).start()
    t01_cp.wait()
    t12_cp.wait()

    do_chunk(0, 0, wait_out=False, start_next=True)
    do_chunk(1, 1, wait_out=False, start_next=True)

    def pair_body(t, carry):
        do_chunk(2 * t, 0, wait_out=True, start_next=True)
        do_chunk(2 * t + 1, 1, wait_out=True, start_next=True)
        return carry

    lax.fori_loop(1, N_PAIRS - 1, pair_body, 0)

    do_chunk(N_CHUNKS - 2, 0, wait_out=True, start_next=False)
    do_chunk(N_CHUNKS - 1, 1, wait_out=True, start_next=False)
    out_copy(N_CHUNKS - 2, 0).wait()
    out_copy(N_CHUNKS - 1, 1).wait()


@jax.jit
def _remap(x, table_01, table_12):
    mesh = plsc.VectorSubcoreMesh(core_axis_name="c", subcore_axis_name="s")
    return pl.kernel(
        _remap_body,
        mesh=mesh,
        out_type=jax.ShapeDtypeStruct((N_ROWS, N_COLS), jnp.int32),
        scratch_types=[
            pltpu.VMEM((T01_SIZE,), jnp.int32),
            pltpu.VMEM((T12_SIZE,), jnp.int32),
            pltpu.VMEM((RCHUNK, N_COLS), jnp.int32),
            pltpu.VMEM((RCHUNK, N_COLS), jnp.int32),
            pltpu.VMEM((RCHUNK, N_COLS), jnp.int32),
            pltpu.VMEM((RCHUNK, N_COLS), jnp.int32),
            pltpu.SemaphoreType.DMA,
            pltpu.SemaphoreType.DMA,
            pltpu.SemaphoreType.DMA,
            pltpu.SemaphoreType.DMA,
            pltpu.SemaphoreType.DMA,
        ],
        compiler_params=pltpu.CompilerParams(needs_layout_passes=False),
    )(x, table_01, table_12)


def kernel(x, table_01, table_12):
    return _remap(x, table_01, table_12)


# speedup vs baseline: 831.9458x; 1.2420x over previous
import jax
import jax.numpy as jnp
from jax import lax
from jax.experimental import pallas as pl
from jax.experimental.pallas import tpu as pltpu
from jax.experimental.pallas import tpu_sc as plsc

L = 16
NC = 2
NS = 16
NW = NC * NS

N_ROWS, N_COLS = 16384, 200
ROWS_PER_W = N_ROWS // NW
RCHUNK = 32
N_CHUNKS = ROWS_PER_W // RCHUNK
N_PAIRS = N_CHUNKS // 2
T01_SIZE = 100000
T12_SIZE = 1000
PK_WORDS = T01_SIZE // 4
W_PER_TILE = 1568
E_PER_TILE = 4 * W_PER_TILE
FUSE_VREGS = W_PER_TILE // L
COL_OFFS = tuple(range(0, 192, 16)) + (N_COLS - L,)


def _remap_body(x_hbm, t01_hbm, t12_hbm, out_hbm, pk_hbm,
                t01s_v, t12_v, pkl_v, pk_v, xb0, xb1, ob0, ob1,
                tsem, is0, is1, os0, os1):
    core = lax.axis_index("c")
    sid = lax.axis_index("s")
    wid = sid * NC + core
    base = wid * ROWS_PER_W
    wb0 = jnp.where(sid == NS - 1, PK_WORDS - W_PER_TILE, sid * W_PER_TILE)

    t01_cp = pltpu.async_copy(
        t01_hbm.at[pl.ds(wb0 * 4, E_PER_TILE)], t01s_v, tsem)
    t12_cp = pltpu.async_copy(t12_hbm, t12_v, tsem)

    xb = [xb0, xb1]
    ob = [ob0, ob1]
    isem = [is0, is1]
    osem = [os0, os1]

    def in_copy(c, b):
        return pltpu.make_async_copy(
            x_hbm.at[pl.ds(base + c * RCHUNK, RCHUNK)], xb[b], isem[b])

    def out_copy(c, b):
        return pltpu.make_async_copy(
            ob[b], out_hbm.at[pl.ds(base + c * RCHUNK, RCHUNK)], osem[b])

    in_copy(0, 0).start()
    in_copy(1, 1).start()
    t01_cp.wait()
    t12_cp.wait()

    lanes4 = lax.iota(jnp.int32, L) * 4

    @plsc.parallel_loop(0, FUSE_VREGS, 1, unroll=2)
    def _fuse(j):
        ebase = j * (4 * L)
        w = None
        for m in range(4):
            v = plsc.load_gather(t01s_v, [lanes4 + (ebase + m)])
            f = plsc.load_gather(t12_v, [v])
            fm = f << (8 * m) if m else f
            w = fm if w is None else w | fm
        pkl_v[pl.ds(j * L, L)] = w

    pltpu.sync_copy(
        pkl_v, pk_hbm.at[pl.ds(core * PK_WORDS + wb0, W_PER_TILE)])
    plsc.subcore_barrier()
    pltpu.sync_copy(pk_hbm.at[pl.ds(core * PK_WORDS, PK_WORDS)], pk_v)

    def compute(b):
        xbc = xb[b]
        obc = ob[b]

        @plsc.parallel_loop(0, RCHUNK, 1, unroll=2)
        def _row(r):
            for c0 in COL_OFFS:
                v = xbc[r, pl.ds(c0, L)]
                w = plsc.load_gather(pk_v, [lax.shift_right_logical(v, 2)])
                sh = (v & 3) << 3
                obc[r, pl.ds(c0, L)] = lax.shift_right_logical(w, sh) & 0xFF

    def pair_body(t, carry):
        for b in (0, 1):
            c = 2 * t + b
            in_copy(c, b).wait()

            @pl.when(c >= 2)
            def _():
                out_copy(c - 2, b).wait()

            compute(b)
            out_copy(c, b).start()

            @pl.when(c + 2 < N_CHUNKS)
            def _():
                in_copy(c + 2, b).start()
        return carry

    lax.fori_loop(0, N_PAIRS, pair_body, 0)

    out_copy(N_CHUNKS - 2, 0).wait()
    out_copy(N_CHUNKS - 1, 1).wait()


@jax.jit
def _remap(x, table_01, table_12):
    mesh = plsc.VectorSubcoreMesh(core_axis_name="c", subcore_axis_name="s")
    out, _ = pl.kernel(
        _remap_body,
        mesh=mesh,
        out_type=(
            jax.ShapeDtypeStruct((N_ROWS, N_COLS), jnp.int32),
            jax.ShapeDtypeStruct((NC * PK_WORDS,), jnp.int32),
        ),
        scratch_types=[
            pltpu.VMEM((E_PER_TILE,), jnp.int32),
            pltpu.VMEM((T12_SIZE,), jnp.int32),
            pltpu.VMEM((W_PER_TILE,), jnp.int32),
            pltpu.VMEM((PK_WORDS,), jnp.int32),
            pltpu.VMEM((RCHUNK, N_COLS), jnp.int32),
            pltpu.VMEM((RCHUNK, N_COLS), jnp.int32),
            pltpu.VMEM((RCHUNK, N_COLS), jnp.int32),
            pltpu.VMEM((RCHUNK, N_COLS), jnp.int32),
            pltpu.SemaphoreType.DMA,
            pltpu.SemaphoreType.DMA,
            pltpu.SemaphoreType.DMA,
            pltpu.SemaphoreType.DMA,
            pltpu.SemaphoreType.DMA,
        ],
        compiler_params=pltpu.CompilerParams(needs_layout_passes=False),
    )(x, table_01, table_12)
    return out


def kernel(x, table_01, table_12):
    return _remap(x, table_01, table_12)


# speedup vs baseline: 1421.3171x; 1.7084x over previous
import jax
import jax.numpy as jnp
from jax import lax
from jax.experimental import pallas as pl
from jax.experimental.pallas import tpu as pltpu
from jax.experimental.pallas import tpu_sc as plsc

L = 16
NC = 2
NS = 16
NW = NC * NS

N_ROWS, N_COLS = 16384, 200
COLS_PER_W = N_ROWS // NW
CCHUNK = 128
N_CHUNKS = COLS_PER_W // CCHUNK
T01_SIZE = 100000
T12_SIZE = 1000
PK_WORDS = T01_SIZE // 4
W_PER_TILE = 1568
E_PER_TILE = 4 * W_PER_TILE
FUSE_VREGS = W_PER_TILE // L
HALF = N_COLS // 2


def _remap_body(x_hbm, t01_hbm, t12_hbm, out_hbm, pk_hbm,
                t01s_v, t12_v, pkl_v, pk_v, bufa, bufb,
                tsem, ia, ib, oa, ob):
    core = lax.axis_index("c")
    sid = lax.axis_index("s")
    wid = sid * NC + core
    cbase = wid * COLS_PER_W
    wb0 = jnp.where(sid == NS - 1, PK_WORDS - W_PER_TILE, sid * W_PER_TILE)

    t01_cp = pltpu.async_copy(
        t01_hbm.at[pl.ds(wb0 * 4, E_PER_TILE)], t01s_v, tsem)
    t12_cp = pltpu.async_copy(t12_hbm, t12_v, tsem)

    buf = [bufa, bufb]
    isem = [ia, ib]
    osem = [oa, ob]

    def in_copy(c, b):
        return pltpu.make_async_copy(
            x_hbm.at[:, pl.ds(cbase + c * CCHUNK, CCHUNK)], buf[b], isem[b])

    def out_copy(c, b):
        return pltpu.make_async_copy(
            buf[b], out_hbm.at[:, pl.ds(cbase + c * CCHUNK, CCHUNK)], osem[b])

    in_copy(0, 0).start()
    in_copy(1, 1).start()
    t01_cp.wait()
    t12_cp.wait()

    lanes4 = lax.iota(jnp.int32, L) * 4

    @plsc.parallel_loop(0, FUSE_VREGS, 1, unroll=2)
    def _fuse(j):
        ebase = j * (4 * L)
        w = None
        for m in range(4):
            v = plsc.load_gather(t01s_v, [lanes4 + (ebase + m)])
            f = plsc.load_gather(t12_v, [v])
            fm = f << (8 * m) if m else f
            w = fm if w is None else w | fm
        pkl_v[pl.ds(j * L, L)] = w

    pltpu.sync_copy(
        pkl_v, pk_hbm.at[pl.ds(core * PK_WORDS + wb0, W_PER_TILE)])
    plsc.subcore_barrier()
    pltpu.sync_copy(pk_hbm.at[pl.ds(core * PK_WORDS, PK_WORDS)], pk_v)

    def compute(b, r0, nrows):
        bc = buf[b]

        @plsc.parallel_loop(r0, r0 + nrows, 1, unroll=2)
        def _row(r):
            for c0 in range(0, CCHUNK, L):
                v = bc[r, pl.ds(c0, L)]
                w = plsc.load_gather(pk_v, [lax.shift_right_logical(v, 2)])
                sh = (v & 3) << 3
                bc[r, pl.ds(c0, L)] = lax.shift_right_logical(w, sh) & 0xFF

    in_copy(0, 0).wait()
    compute(0, 0, N_COLS)
    out_copy(0, 0).start()
    in_copy(1, 1).wait()
    compute(1, 0, HALF)
    out_copy(0, 0).wait()
    in_copy(2, 0).start()
    compute(1, HALF, N_COLS - HALF)
    out_copy(1, 1).start()
    in_copy(2, 0).wait()
    compute(0, 0, HALF)
    out_copy(1, 1).wait()
    in_copy(3, 1).start()
    compute(0, HALF, N_COLS - HALF)
    out_copy(2, 0).start()
    in_copy(3, 1).wait()
    compute(1, 0, N_COLS)
    out_copy(3, 1).start()

    out_copy(2, 0).wait()
    out_copy(3, 1).wait()


@jax.jit
def _remap(xt, table_01, table_12):
    mesh = plsc.VectorSubcoreMesh(core_axis_name="c", subcore_axis_name="s")
    out_t, _ = pl.kernel(
        _remap_body,
        mesh=mesh,
        out_type=(
            jax.ShapeDtypeStruct((N_COLS, N_ROWS), jnp.int32),
            jax.ShapeDtypeStruct((NC * PK_WORDS,), jnp.int32),
        ),
        scratch_types=[
            pltpu.VMEM((E_PER_TILE,), jnp.int32),
            pltpu.VMEM((T12_SIZE,), jnp.int32),
            pltpu.VMEM((W_PER_TILE,), jnp.int32),
            pltpu.VMEM((PK_WORDS,), jnp.int32),
            pltpu.VMEM((N_COLS, CCHUNK), jnp.int32),
            pltpu.VMEM((N_COLS, CCHUNK), jnp.int32),
            pltpu.SemaphoreType.DMA,
            pltpu.SemaphoreType.DMA,
            pltpu.SemaphoreType.DMA,
            pltpu.SemaphoreType.DMA,
            pltpu.SemaphoreType.DMA,
        ],
        compiler_params=pltpu.CompilerParams(needs_layout_passes=False),
    )(xt, table_01, table_12)
    return out_t


def kernel(x, table_01, table_12):
    return _remap(x.T, table_01, table_12).T


# speedup vs baseline: 1426.2977x; 1.0035x over previous
import jax
import jax.numpy as jnp
from jax import lax
from jax.experimental import pallas as pl
from jax.experimental.pallas import tpu as pltpu
from jax.experimental.pallas import tpu_sc as plsc

L = 16
NC = 2
NS = 16
NW = NC * NS

N_ROWS, N_COLS = 16384, 200
COLS_PER_W = N_ROWS // NW
CCHUNK = 128
N_CHUNKS = COLS_PER_W // CCHUNK
T01_SIZE = 100000
T12_SIZE = 1000
PK_WORDS = T01_SIZE // 4
W_PER_TILE = 1568
E_PER_TILE = 4 * W_PER_TILE
FUSE_VREGS = W_PER_TILE // L
HALF = N_COLS // 2


def _remap_body(x_hbm, t01_hbm, t12_hbm, out_hbm, pk_hbm,
                t01s_v, t12_v, pkl_v, pk_v, bufa, bufb,
                tsem, ia, ib, oa, ob):
    core = lax.axis_index("c")
    sid = lax.axis_index("s")
    wid = sid * NC + core
    cbase = wid * COLS_PER_W
    wb0 = jnp.where(sid == NS - 1, PK_WORDS - W_PER_TILE, sid * W_PER_TILE)

    t01_cp = pltpu.async_copy(
        t01_hbm.at[pl.ds(wb0 * 4, E_PER_TILE)], t01s_v, tsem)
    t12_cp = pltpu.async_copy(t12_hbm, t12_v, tsem)

    buf = [bufa, bufb]
    isem = [ia, ib]
    osem = [oa, ob]

    def in_copy(c, b):
        return pltpu.make_async_copy(
            x_hbm.at[:, pl.ds(cbase + c * CCHUNK, CCHUNK)], buf[b], isem[b])

    def out_copy(c, b):
        return pltpu.make_async_copy(
            buf[b], out_hbm.at[:, pl.ds(cbase + c * CCHUNK, CCHUNK)], osem[b])

    in_copy(0, 0).start()
    in_copy(1, 1).start()
    t01_cp.wait()
    t12_cp.wait()

    lanes4 = lax.iota(jnp.int32, L) * 4

    @plsc.parallel_loop(0, FUSE_VREGS, 1, unroll=2)
    def _fuse(j):
        ebase = j * (4 * L)
        w = None
        for m in range(4):
            v = plsc.load_gather(t01s_v, [lanes4 + (ebase + m)])
            f = plsc.load_gather(t12_v, [v])
            fm = f << (8 * m) if m else f
            w = fm if w is None else w | fm
        pkl_v[pl.ds(j * L, L)] = w

    pltpu.sync_copy(
        pkl_v, pk_hbm.at[pl.ds(core * PK_WORDS + wb0, W_PER_TILE)])
    plsc.subcore_barrier()
    pltpu.sync_copy(pk_hbm.at[pl.ds(core * PK_WORDS, PK_WORDS)], pk_v)

    def compute(b, r0, nrows):
        bc = buf[b]

        @plsc.parallel_loop(r0, r0 + nrows, 1, unroll=2)
        def _row(r):
            for c0 in range(0, CCHUNK, L):
                v = bc[r, pl.ds(c0, L)]
                w = plsc.load_gather(pk_v, [lax.shift_right_logical(v, 2)])
                sh = (v & 3) << 3
                bc[r, pl.ds(c0, L)] = lax.shift_right_logical(w, sh) & 0xFF

    in_copy(0, 0).wait()
    compute(0, 0, N_COLS)
    out_copy(0, 0).start()
    in_copy(1, 1).wait()
    compute(1, 0, HALF)
    out_copy(0, 0).wait()
    in_copy(2, 0).start()
    compute(1, HALF, N_COLS - HALF)
    out_copy(1, 1).start()
    in_copy(2, 0).wait()
    compute(0, 0, HALF)
    out_copy(1, 1).wait()
    in_copy(3, 1).start()
    compute(0, HALF, N_COLS - HALF)
    out_copy(2, 0).start()
    in_copy(3, 1).wait()
    compute(1, 0, N_COLS)
    out_copy(3, 1).start()

    out_copy(2, 0).wait()
    out_copy(3, 1).wait()


@jax.jit
def _remap(xt, table_01, table_12):
    mesh = plsc.VectorSubcoreMesh(core_axis_name="c", subcore_axis_name="s")
    out_t, _ = pl.kernel(
        _remap_body,
        mesh=mesh,
        out_type=(
            jax.ShapeDtypeStruct((N_COLS, N_ROWS), jnp.int32),
            jax.ShapeDtypeStruct((NC * PK_WORDS,), jnp.int32),
        ),
        scratch_types=[
            pltpu.VMEM((E_PER_TILE,), jnp.int32),
            pltpu.VMEM((T12_SIZE,), jnp.int32),
            pltpu.VMEM((W_PER_TILE,), jnp.int32),
            pltpu.VMEM((PK_WORDS,), jnp.int32),
            pltpu.VMEM((N_COLS, CCHUNK), jnp.int32),
            pltpu.VMEM((N_COLS, CCHUNK), jnp.int32),
            pltpu.SemaphoreType.DMA,
            pltpu.SemaphoreType.DMA,
            pltpu.SemaphoreType.DMA,
            pltpu.SemaphoreType.DMA,
            pltpu.SemaphoreType.DMA,
        ],
        compiler_params=pltpu.CompilerParams(
            needs_layout_passes=False, skip_device_barrier=True),
    )(xt, table_01, table_12)
    return out_t


def kernel(x, table_01, table_12):
    return _remap(x.T, table_01, table_12).T
